# Initial kernel scaffold; baseline (speedup 1.0000x reference)
import jax
import jax.numpy as jnp
from jax.experimental import pallas as pl


def kernel(t, x, h_init, emb_w, emb_b, edge_w1, edge_b1, edge_w2, edge_b2, att_w, att_b, node_w1, node_b1, node_w2, node_b2, coord_w1, coord_b1, coord_w2, rows, cols):
    raise NotImplementedError("write your pallas kernel here")


# speedup vs baseline: 11.9506x; 11.9506x over previous
import functools

import jax
import jax.numpy as jnp
from jax.experimental import pallas as pl
from jax.experimental.pallas import tpu as pltpu

NPV = 22
PP = 24
ND = 3
HID = 64
NL = 5
HSZ = 21
CRANGE = 15.0


def _silu(v):
    return v * jax.nn.sigmoid(v)


def _egnn_block(t_ref, x_ref, hinit_ref, emb_w_ref, emb_b_ref, ew1_ref,
                eb1_ref, ew2_ref, eb2_ref, aw_ref, ab_ref, nw1_ref, nb1_ref,
                nw2_ref, nb2_ref, cw1_ref, cb1_ref, cw2_ref, out_ref, *, bm):
    f32 = jnp.float32
    jj = jax.lax.broadcasted_iota(jnp.int32, (1, PP, PP), 1)
    ii = jax.lax.broadcasted_iota(jnp.int32, (1, PP, PP), 2)
    smask = ((jj != ii) & (jj < NPV)).astype(f32)
    jj4 = jax.lax.broadcasted_iota(jnp.int32, (1, PP, PP, 1), 1)
    ii4 = jax.lax.broadcasted_iota(jnp.int32, (1, PP, PP, 1), 2)
    aggmask = ((jj4 != ii4) & (jj4 < NPV)).astype(f32)
    lane_valid = (jax.lax.broadcasted_iota(jnp.int32, (1, PP), 1)
                  < NPV).astype(f32)

    xs = [x_ref[d] for d in range(ND)]
    x0s = list(xs)

    base = jnp.dot(hinit_ref[...], emb_w_ref[0:HSZ, :],
                   preferred_element_type=f32) + emb_b_ref[...]
    base = jnp.concatenate(
        [base, jnp.zeros((PP - NPV, HID), f32)], axis=0)
    tcol = t_ref[...]
    H3 = base[None] + tcol[:, :, None] * emb_w_ref[HSZ:HSZ + 1, :][None]
    H = H3.reshape(bm * PP, HID)

    def radial3(xds):
        r = jnp.zeros((bm, PP, PP), f32)
        for d in range(ND):
            xi = jnp.broadcast_to(xds[d][:, None, :], (bm, PP, PP))
            xj = jnp.broadcast_to(xds[d][:, :, None], (bm, PP, PP))
            df = xi - xj
            r = r + df * df
        return r

    eattr4 = radial3(x0s).reshape(bm, PP, PP, 1)

    crl = CRANGE / NL
    for l in range(NL):
        w1 = ew1_ref[l]
        W1a = w1[0:HID, :]
        W1b = w1[HID:2 * HID, :]
        wr = w1[2 * HID:2 * HID + 1, :]
        we = w1[2 * HID + 1:2 * HID + 2, :]
        A = jnp.dot(H, W1a, preferred_element_type=f32)
        B = jnp.dot(H, W1b, preferred_element_type=f32)
        A3 = A.reshape(bm, PP, HID)
        B3 = B.reshape(bm, PP, HID)
        A4 = jnp.broadcast_to(A3[:, None, :, :], (bm, PP, PP, HID))
        B4 = jnp.broadcast_to(B3[:, :, None, :], (bm, PP, PP, HID))
        r4 = radial3(xs).reshape(bm, PP, PP, 1)
        e4 = (A4 + B4 + r4 * wr[None, None] + eattr4 * we[None, None]
              + eb1_ref[l:l + 1, :][None, None])
        E = e4.reshape(bm * PP * PP, HID)
        m1 = _silu(E)
        m2 = _silu(jnp.dot(m1, ew2_ref[l], preferred_element_type=f32)
                   + eb2_ref[l:l + 1, :])
        gate = jax.nn.sigmoid(
            jnp.dot(m2, aw_ref[l], preferred_element_type=f32)
            + ab_ref[l:l + 1, :])
        m3 = m2 * gate
        phi = _silu(jnp.dot(m3, cw1_ref[l], preferred_element_type=f32)
                    + cb1_ref[l:l + 1, :])
        s = jnp.tanh(jnp.dot(phi, cw2_ref[l],
                             preferred_element_type=f32)) * crl
        s3 = s.reshape(bm, PP, PP) * smask
        ssum = jnp.sum(s3, axis=1)
        new_xs = []
        for d in range(ND):
            xj = jnp.broadcast_to(xs[d][:, :, None], (bm, PP, PP))
            sx = jnp.sum(s3 * xj, axis=1)
            new_xs.append(xs[d] + xs[d] * ssum - sx)
        xs = new_xs
        m3m = m3.reshape(bm, PP, PP, HID) * aggmask
        agg = jnp.sum(m3m, axis=1).reshape(bm * PP, HID)
        hin = jnp.concatenate([H, agg], axis=1)
        hn = _silu(jnp.dot(hin, nw1_ref[l], preferred_element_type=f32)
                   + nb1_ref[l:l + 1, :])
        H = H + jnp.dot(hn, nw2_ref[l], preferred_element_type=f32) \
            + nb2_ref[l:l + 1, :]

    for d in range(ND):
        veld = (xs[d] - x0s[d]) * lane_valid
        mean = jnp.sum(veld, axis=1, keepdims=True) * (1.0 / NPV)
        out_ref[d] = (veld - mean) * lane_valid


def kernel(t, x, h_init, emb_w, emb_b, edge_w1, edge_b1, edge_w2, edge_b2,
           att_w, att_b, node_w1, node_b1, node_w2, node_b2, coord_w1,
           coord_b1, coord_w2, rows, cols):
    del rows, cols
    nb = x.shape[0]
    bm = 8
    xt = x.reshape(nb, NPV, ND).transpose(2, 0, 1)
    xt = jnp.pad(xt, ((0, 0), (0, 0), (0, PP - NPV)))
    t2 = t.reshape(nb, 1)
    emb_b2d = emb_b.reshape(1, HID)

    def full(a):
        return pl.BlockSpec(a.shape, lambda b: (0,) * a.ndim)

    weights = (h_init, emb_w, emb_b2d, edge_w1, edge_b1, edge_w2, edge_b2,
               att_w, att_b, node_w1, node_b1, node_w2, node_b2, coord_w1,
               coord_b1, coord_w2)
    out = pl.pallas_call(
        functools.partial(_egnn_block, bm=bm),
        grid=(nb // bm,),
        in_specs=[
            pl.BlockSpec((bm, 1), lambda b: (b, 0)),
            pl.BlockSpec((ND, bm, PP), lambda b: (0, b, 0)),
        ] + [full(a) for a in weights],
        out_specs=pl.BlockSpec((ND, bm, PP), lambda b: (0, b, 0)),
        out_shape=jax.ShapeDtypeStruct((ND, nb, PP), jnp.float32),
        compiler_params=pltpu.CompilerParams(
            dimension_semantics=("arbitrary",)),
    )(t2, xt, *weights)
    vel = out.transpose(1, 2, 0)[:, :NPV, :].reshape(nb, NPV * ND)
    return vel


# speedup vs baseline: 14.2615x; 1.1934x over previous
import functools

import jax
import jax.numpy as jnp
from jax.experimental import pallas as pl
from jax.experimental.pallas import tpu as pltpu

NPV = 22
PP = 24
QJ = 4
JH = PP // QJ
ND = 3
HID = 64
NL = 5
HSZ = 21
CRANGE = 15.0


def _silu(v):
    return v * jax.nn.sigmoid(v)


def _egnn_block(t_ref, x_ref, hinit_ref, embw_ref, embb_ref, w1ab_ref,
                wr_ref, we_ref, eb1_ref, w2bd_ref, eb2_ref, awk_ref, ab_ref,
                cw1bd_ref, cb1_ref, cw2k_ref, nw1_ref, nb1_ref, nw2_ref,
                nb2_ref, out_ref, *, bm):
    f32 = jnp.float32
    R = bm * JH * PP
    NN = bm * PP

    jh4 = jax.lax.broadcasted_iota(jnp.int32, (1, JH, PP, 256), 1)
    ii4 = jax.lax.broadcasted_iota(jnp.int32, (1, JH, PP, 256), 2)
    ll4 = jax.lax.broadcasted_iota(jnp.int32, (1, JH, PP, 256), 3)
    jfull = jh4 * QJ + ll4 // HID
    aggmask = ((jfull != ii4) & (jfull < NPV)).astype(f32)
    jhs = jax.lax.broadcasted_iota(jnp.int32, (1, JH, PP, QJ), 1)
    iis = jax.lax.broadcasted_iota(jnp.int32, (1, JH, PP, QJ), 2)
    lls = jax.lax.broadcasted_iota(jnp.int32, (1, JH, PP, QJ), 3)
    js = jhs * QJ + lls
    smask = ((js != iis) & (js < NPV)).astype(f32)
    nodemask = (jax.lax.broadcasted_iota(jnp.int32, (1, PP, 1), 1)
                < NPV).astype(f32)

    xcols = [x_ref[:, d:d + 1].reshape(bm, PP, 1) for d in range(ND)]
    x0cols = list(xcols)

    def sender_quads(xc):
        return jnp.swapaxes(xc.reshape(bm, JH, QJ, 1), 2, 3)

    def radial_full(xcs):
        r = jnp.zeros((bm, JH, PP, 256), f32)
        for d in range(ND):
            xi = xcs[d][:, None]
            xq = sender_quads(xcs[d])
            xj = jnp.concatenate(
                [jnp.broadcast_to(xq[:, :, :, q:q + 1], (bm, JH, 1, HID))
                 for q in range(QJ)], axis=3)
            df = xi - xj
            r = r + df * df
        return r

    base = jnp.dot(hinit_ref[...], embw_ref[0:HSZ, :],
                   preferred_element_type=f32) + embb_ref[...]
    base = jnp.concatenate(
        [base, jnp.zeros((PP - NPV, HID), f32)], axis=0)
    H3 = base[None] + t_ref[...][:, :, None] * embw_ref[HSZ:HSZ + 1, :][None]
    H = H3.reshape(NN, HID)

    eattr4 = radial_full(x0cols)

    crl = CRANGE / NL
    for l in range(NL):
        AB = jnp.dot(H, w1ab_ref[l], preferred_element_type=f32)
        A3 = AB[:, 0:HID].reshape(bm, PP, HID)
        B3 = AB[:, HID:2 * HID].reshape(bm, PP, HID)
        A4 = jnp.concatenate([A3] * QJ, axis=2)[:, None]
        Bq = B3.reshape(bm, JH, QJ, HID)
        Bj = jnp.concatenate(
            [Bq[:, :, q, :][:, :, None, :] for q in range(QJ)],
            axis=3)
        r4 = radial_full(xcols)
        e4 = (A4 + Bj + r4 * wr_ref[l:l + 1, :][None, None]
              + eattr4 * we_ref[l:l + 1, :][None, None]
              + eb1_ref[l:l + 1, :][None, None])
        E = e4.reshape(R, 4 * HID)
        m1 = _silu(E)
        m2 = _silu(jnp.dot(m1, w2bd_ref[l], preferred_element_type=f32)
                   + eb2_ref[l:l + 1, :])
        g4 = jax.nn.sigmoid(
            jnp.dot(m2, awk_ref[l], preferred_element_type=f32)
            + ab_ref[l:l + 1, :])
        gate = jnp.concatenate(
            [jnp.broadcast_to(g4[:, q:q + 1], (R, HID)) for q in range(QJ)],
            axis=1)
        m3 = m2 * gate
        phi = _silu(jnp.dot(m3, cw1bd_ref[l], preferred_element_type=f32)
                    + cb1_ref[l:l + 1, :])
        sp = jnp.tanh(jnp.dot(phi, cw2k_ref[l],
                              preferred_element_type=f32)) * crl
        s3 = sp.reshape(bm, JH, PP, QJ) * smask
        ssum = jnp.sum(jnp.sum(s3, axis=3, keepdims=True), axis=1)
        new_xcols = []
        for d in range(ND):
            xq = sender_quads(xcols[d])
            sx = jnp.sum(jnp.sum(s3 * xq, axis=3, keepdims=True), axis=1)
            new_xcols.append(xcols[d] + xcols[d] * ssum - sx)
        m3m = e4_agg = m3.reshape(bm, JH, PP, 4 * HID) * aggmask
        aggq = jnp.sum(m3m, axis=1)
        agg = (aggq[:, :, 0:HID] + aggq[:, :, HID:2 * HID]
               + aggq[:, :, 2 * HID:3 * HID]
               + aggq[:, :, 3 * HID:4 * HID]).reshape(NN, HID)
        xcols = new_xcols
        hin = jnp.concatenate([H, agg], axis=1)
        hn = _silu(jnp.dot(hin, nw1_ref[l], preferred_element_type=f32)
                   + nb1_ref[l:l + 1, :])
        H = H + jnp.dot(hn, nw2_ref[l], preferred_element_type=f32) \
            + nb2_ref[l:l + 1, :]

    for d in range(ND):
        veld = (xcols[d] - x0cols[d]) * nodemask
        mean = jnp.sum(veld, axis=1, keepdims=True) * (1.0 / NPV)
        out_ref[:, d:d + 1] = ((veld - mean) * nodemask).reshape(NN, 1)


def kernel(t, x, h_init, emb_w, emb_b, edge_w1, edge_b1, edge_w2, edge_b2,
           att_w, att_b, node_w1, node_b1, node_w2, node_b2, coord_w1,
           coord_b1, coord_w2, rows, cols):
    del rows, cols
    nb = x.shape[0]
    bm = 8
    f32 = jnp.float32

    xp = jnp.pad(x.reshape(nb, NPV, ND), ((0, 0), (0, PP - NPV), (0, 0)))
    xp = xp.reshape(nb * PP, ND)
    t2 = t.reshape(nb, 1)

    eye4 = jnp.eye(QJ, dtype=f32)
    w1ab = jnp.concatenate(
        [edge_w1[:, 0:HID, :], edge_w1[:, HID:2 * HID, :]],
        axis=2)
    wr_t = jnp.tile(edge_w1[:, 2 * HID, :], (1, QJ))
    we_t = jnp.tile(edge_w1[:, 2 * HID + 1, :], (1, QJ))
    eb1_t = jnp.tile(edge_b1, (1, QJ))
    eb2_t = jnp.tile(edge_b2, (1, QJ))
    cb1_t = jnp.tile(coord_b1, (1, QJ))
    w2bd = jax.vmap(lambda w: jnp.kron(eye4, w))(edge_w2)
    cw1bd = jax.vmap(lambda w: jnp.kron(eye4, w))(coord_w1)
    awk = jax.vmap(lambda w: jnp.kron(eye4, w))(att_w)
    cw2k = jax.vmap(lambda w: jnp.kron(eye4, w))(coord_w2)
    embb2 = emb_b.reshape(1, HID)

    def full(a):
        return pl.BlockSpec(a.shape, lambda b: (0,) * a.ndim)

    weights = (h_init, emb_w, embb2, w1ab, wr_t, we_t, eb1_t, w2bd, eb2_t,
               awk, att_b, cw1bd, cb1_t, cw2k, node_w1, node_b1, node_w2,
               node_b2)
    out = pl.pallas_call(
        functools.partial(_egnn_block, bm=bm),
        grid=(nb // bm,),
        in_specs=[
            pl.BlockSpec((bm, 1), lambda b: (b, 0)),
            pl.BlockSpec((bm * PP, ND), lambda b: (b, 0)),
        ] + [full(a) for a in weights],
        out_specs=pl.BlockSpec((bm * PP, ND), lambda b: (b, 0)),
        out_shape=jax.ShapeDtypeStruct((nb * PP, ND), f32),
        compiler_params=pltpu.CompilerParams(
            dimension_semantics=("arbitrary",)),
    )(t2, xp, *weights)
    vel = out.reshape(nb, PP, ND)[:, :NPV, :].reshape(nb, NPV * ND)
    return vel


# speedup vs baseline: 17.2378x; 1.2087x over previous
import functools

import jax
import jax.numpy as jnp
from jax.experimental import pallas as pl
from jax.experimental.pallas import tpu as pltpu

NPV = 22
PP = 24
QJ = 4
JH = PP // QJ
ND = 3
HID = 64
NL = 5
HSZ = 21
CRANGE = 15.0


def _silu(v):
    return v * jax.nn.sigmoid(v)


def _egnn_block(t_ref, x_ref, hinit_ref, embw_ref, embb_ref, w1ab_ref,
                wr_ref, we_ref, eb1_ref, w2bd_ref, eb2_ref, awk_ref, ab_ref,
                cw1bd_ref, cb1_ref, cw2k_ref, nw1_ref, nb1_ref, nw2_ref,
                nb2_ref, out_ref, *, bm):
    f32 = jnp.float32
    R = bm * JH * PP
    NN = bm * PP

    jh4 = jax.lax.broadcasted_iota(jnp.int32, (1, JH, PP, 256), 1)
    ii4 = jax.lax.broadcasted_iota(jnp.int32, (1, JH, PP, 256), 2)
    ll4 = jax.lax.broadcasted_iota(jnp.int32, (1, JH, PP, 256), 3)
    jfull = jh4 * QJ + ll4 // HID
    aggmask = ((jfull != ii4) & (jfull < NPV)).astype(f32)
    jhs = jax.lax.broadcasted_iota(jnp.int32, (1, JH, PP, QJ), 1)
    iis = jax.lax.broadcasted_iota(jnp.int32, (1, JH, PP, QJ), 2)
    lls = jax.lax.broadcasted_iota(jnp.int32, (1, JH, PP, QJ), 3)
    js = jhs * QJ + lls
    smask = ((js != iis) & (js < NPV)).astype(f32)
    nodemask = (jax.lax.broadcasted_iota(jnp.int32, (1, PP, 1), 1)
                < NPV).astype(f32)

    xcols = [x_ref[:, d:d + 1].reshape(bm, PP, 1) for d in range(ND)]
    x0cols = list(xcols)

    def sender_quads(xc):
        return jnp.swapaxes(xc.reshape(bm, JH, QJ, 1), 2, 3)

    def pair_diffs(xcs):
        return [xcs[d][:, None] - jnp.broadcast_to(
            sender_quads(xcs[d]), (bm, JH, PP, QJ)) for d in range(ND)]

    def quad_widen(v):
        return jnp.concatenate(
            [jnp.broadcast_to(v[:, :, :, q:q + 1], (bm, JH, PP, HID))
             for q in range(QJ)], axis=3)

    def radial_s(dfs):
        return dfs[0] * dfs[0] + dfs[1] * dfs[1] + dfs[2] * dfs[2]

    def rb(v):
        return v.astype(jnp.bfloat16).astype(f32)

    base = jnp.dot(hinit_ref[...], embw_ref[0:HSZ, :],
                   preferred_element_type=f32) + embb_ref[...]
    base = jnp.concatenate(
        [base, jnp.zeros((PP - NPV, HID), f32)], axis=0)
    H3 = base[None] + rb(t_ref[...][:, :, None]) \
        * rb(embw_ref[HSZ:HSZ + 1, :][None])
    H = H3.reshape(NN, HID)

    eattr4 = quad_widen(rb(radial_s(pair_diffs(x0cols))))

    crl = CRANGE / NL
    for l in range(NL):
        AB = jnp.dot(H, w1ab_ref[l], preferred_element_type=f32)
        A3 = AB[:, 0:HID].reshape(bm, PP, HID)
        B3 = AB[:, HID:2 * HID].reshape(bm, PP, HID)
        A4 = jnp.concatenate([A3] * QJ, axis=2)[:, None]
        Bq = B3.reshape(bm, JH, QJ, HID)
        Bj = jnp.concatenate(
            [Bq[:, :, q, :][:, :, None, :] for q in range(QJ)],
            axis=3)
        dfs = pair_diffs(xcols)
        r4 = quad_widen(rb(radial_s(dfs)))
        e4 = (A4 + Bj + r4 * rb(wr_ref[l:l + 1, :][None, None])
              + eattr4 * rb(we_ref[l:l + 1, :][None, None])
              + eb1_ref[l:l + 1, :][None, None])
        E = e4.reshape(R, 4 * HID)
        m1 = _silu(E)
        m2 = _silu(jnp.dot(m1, w2bd_ref[l], preferred_element_type=f32)
                   + eb2_ref[l:l + 1, :])
        g4 = jax.nn.sigmoid(
            jnp.dot(m2, awk_ref[l], preferred_element_type=f32)
            + ab_ref[l:l + 1, :])
        gate = jnp.concatenate(
            [jnp.broadcast_to(g4[:, q:q + 1], (R, HID)) for q in range(QJ)],
            axis=1)
        m3 = m2 * gate
        phi = _silu(jnp.dot(m3, cw1bd_ref[l], preferred_element_type=f32)
                    + cb1_ref[l:l + 1, :])
        sp = jnp.tanh(jnp.dot(phi, cw2k_ref[l],
                              preferred_element_type=f32)) * crl
        s3 = sp.reshape(bm, JH, PP, QJ) * smask
        new_xcols = []
        for d in range(ND):
            sx = jnp.sum(jnp.sum(s3 * dfs[d], axis=3, keepdims=True), axis=1)
            new_xcols.append(xcols[d] + sx)
        m3m = e4_agg = m3.reshape(bm, JH, PP, 4 * HID) * aggmask
        aggq = jnp.sum(m3m, axis=1)
        agg = (aggq[:, :, 0:HID] + aggq[:, :, HID:2 * HID]
               + aggq[:, :, 2 * HID:3 * HID]
               + aggq[:, :, 3 * HID:4 * HID]).reshape(NN, HID)
        xcols = new_xcols
        hin = jnp.concatenate([H, agg], axis=1)
        hn = _silu(jnp.dot(hin, nw1_ref[l], preferred_element_type=f32)
                   + nb1_ref[l:l + 1, :])
        H = H + jnp.dot(hn, nw2_ref[l], preferred_element_type=f32) \
            + nb2_ref[l:l + 1, :]

    for d in range(ND):
        veld = (xcols[d] - x0cols[d]) * nodemask
        mean = jnp.sum(veld, axis=1, keepdims=True) * (1.0 / NPV)
        out_ref[:, d:d + 1] = ((veld - mean) * nodemask).reshape(NN, 1)


def kernel(t, x, h_init, emb_w, emb_b, edge_w1, edge_b1, edge_w2, edge_b2,
           att_w, att_b, node_w1, node_b1, node_w2, node_b2, coord_w1,
           coord_b1, coord_w2, rows, cols):
    del rows, cols
    nb = x.shape[0]
    bm = 8
    f32 = jnp.float32

    xp = jnp.pad(x.reshape(nb, NPV, ND), ((0, 0), (0, PP - NPV), (0, 0)))
    xp = xp.reshape(nb * PP, ND)
    t2 = t.reshape(nb, 1)

    eye4 = jnp.eye(QJ, dtype=f32)
    w1ab = jnp.concatenate(
        [edge_w1[:, 0:HID, :], edge_w1[:, HID:2 * HID, :]],
        axis=2)
    wr_t = jnp.tile(edge_w1[:, 2 * HID, :], (1, QJ))
    we_t = jnp.tile(edge_w1[:, 2 * HID + 1, :], (1, QJ))
    eb1_t = jnp.tile(edge_b1, (1, QJ))
    eb2_t = jnp.tile(edge_b2, (1, QJ))
    cb1_t = jnp.tile(coord_b1, (1, QJ))
    w2bd = jax.vmap(lambda w: jnp.kron(eye4, w))(edge_w2)
    cw1bd = jax.vmap(lambda w: jnp.kron(eye4, w))(coord_w1)
    awk = jax.vmap(lambda w: jnp.kron(eye4, w))(att_w)
    cw2k = jax.vmap(lambda w: jnp.kron(eye4, w))(coord_w2)
    embb2 = emb_b.reshape(1, HID)

    def full(a):
        return pl.BlockSpec(a.shape, lambda b: (0,) * a.ndim)

    weights = (h_init, emb_w, embb2, w1ab, wr_t, we_t, eb1_t, w2bd, eb2_t,
               awk, att_b, cw1bd, cb1_t, cw2k, node_w1, node_b1, node_w2,
               node_b2)
    out = pl.pallas_call(
        functools.partial(_egnn_block, bm=bm),
        grid=(nb // bm,),
        in_specs=[
            pl.BlockSpec((bm, 1), lambda b: (b, 0)),
            pl.BlockSpec((bm * PP, ND), lambda b: (b, 0)),
        ] + [full(a) for a in weights],
        out_specs=pl.BlockSpec((bm * PP, ND), lambda b: (b, 0)),
        out_shape=jax.ShapeDtypeStruct((nb * PP, ND), f32),
        compiler_params=pltpu.CompilerParams(
            dimension_semantics=("arbitrary",)),
    )(t2, xp, *weights)
    vel = out.reshape(nb, PP, ND)[:, :NPV, :].reshape(nb, NPV * ND)
    return vel
